# baseline (speedup 1.0000x reference)
import functools

import jax
import jax.numpy as jnp
from jax import lax
from jax.experimental import pallas as pl
from jax.experimental.pallas import tpu as pltpu
from jax.experimental.pallas import tpu_sc as plsc

_CHUNK = 16


@functools.lru_cache(maxsize=None)
def _permute_rows(R, D, C, dtype_name):
    dtype = jnp.dtype(dtype_name)
    info = plsc.get_sparse_core_info()
    NC, NS = info.num_cores, info.num_subcores
    NW = NC * NS
    assert R % (NW * _CHUNK) == 0
    assert C % _CHUNK == 0
    rows_per_w = R // NW
    n_chunks = rows_per_w // _CHUNK
    mesh = plsc.VectorSubcoreMesh(core_axis_name="c", subcore_axis_name="s")

    @functools.partial(
        pl.kernel,
        mesh=mesh,
        out_type=jax.ShapeDtypeStruct((R, D), dtype),
        compiler_params=pltpu.CompilerParams(use_tc_tiling_on_sc=False),
        scratch_types=[
            pltpu.VMEM((C,), jnp.int32),
            pltpu.VMEM((_CHUNK, D), dtype),
            pltpu.SemaphoreType.DMA,
        ],
    )
    def k(x_hbm, perm_hbm, out_hbm, perm_v, buf, sem):
        wid = lax.axis_index("s") * NC + lax.axis_index("c")
        base = wid * rows_per_w
        pltpu.sync_copy(perm_hbm, perm_v)

        def step(i, carry):
            r0 = base + i * _CHUNK
            c0 = lax.rem(r0, C)
            pvec = perm_v[pl.ds(c0, _CHUNK)]
            gidx = (r0 - c0) + pvec
            pltpu.async_copy(x_hbm.at[gidx], buf, sem).wait()
            pltpu.sync_copy(buf, out_hbm.at[pl.ds(r0, _CHUNK)])
            return carry

        lax.fori_loop(0, n_chunks, step, 0)

    return k


def kernel(x, permutation):
    b, c, h, w = x.shape
    D = h * w
    x2 = x.reshape(b * c, D)
    z2 = _permute_rows(b * c, D, c, x.dtype.name)(x2, permutation)
    z = z2.reshape(b, c, h, w)
    ldj = jnp.zeros((b,), x.dtype)
    return (z, ldj)


# speedup vs baseline: 2.4091x; 2.4091x over previous
import functools

import jax
import jax.numpy as jnp
from jax import lax
from jax.experimental import pallas as pl
from jax.experimental.pallas import tpu as pltpu
from jax.experimental.pallas import tpu_sc as plsc

_LANES = 16
_RB = 64


@functools.lru_cache(maxsize=None)
def _permute_cols(R, C, dtype_name):
    dtype = jnp.dtype(dtype_name)
    info = plsc.get_sparse_core_info()
    NC, NS = info.num_cores, info.num_subcores
    NW = NC * NS
    assert R % (NW * _RB) == 0
    assert C % _LANES == 0
    rows_per_w = R // NW
    n_blocks = rows_per_w // _RB
    n_cgrp = C // _LANES
    mesh = plsc.VectorSubcoreMesh(core_axis_name="c", subcore_axis_name="s")

    @functools.partial(
        pl.kernel,
        mesh=mesh,
        out_type=jax.ShapeDtypeStruct((R, C), dtype),
        compiler_params=pltpu.CompilerParams(needs_layout_passes=False),
        scratch_types=[
            pltpu.VMEM((C,), jnp.int32),
            pltpu.VMEM((_RB, C), dtype),
            pltpu.VMEM((_RB, C), dtype),
        ],
    )
    def k(x_hbm, perm_hbm, out_hbm, perm_v, ibuf, obuf):
        wid = lax.axis_index("s") * NC + lax.axis_index("c")
        base = wid * rows_per_w
        pltpu.sync_copy(perm_hbm, perm_v)
        pvecs = [perm_v[pl.ds(j * _LANES, _LANES)] for j in range(n_cgrp)]

        def blk(ib, carry):
            r0 = base + ib * _RB
            pltpu.sync_copy(x_hbm.at[pl.ds(r0, _RB)], ibuf)

            def row(r, c2):
                for j in range(n_cgrp):
                    rvec = jnp.broadcast_to(r, (_LANES,)).astype(jnp.int32)
                    g = plsc.load_gather(ibuf, [rvec, pvecs[j]])
                    obuf[r, pl.ds(j * _LANES, _LANES)] = g
                return c2

            lax.fori_loop(0, _RB, row, 0)
            pltpu.sync_copy(obuf, out_hbm.at[pl.ds(r0, _RB)])
            return carry

        lax.fori_loop(0, n_blocks, blk, 0)

    return k


def kernel(x, permutation):
    b, c, h, w = x.shape
    xt = jnp.transpose(x, (0, 2, 3, 1)).reshape(b * h * w, c)
    zt = _permute_cols(b * h * w, c, x.dtype.name)(xt, permutation)
    z = jnp.transpose(zt.reshape(b, h, w, c), (0, 3, 1, 2))
    ldj = jnp.zeros((b,), x.dtype)
    return (z, ldj)


# speedup vs baseline: 3.8998x; 1.6188x over previous
import functools

import jax
import jax.numpy as jnp
from jax import lax
from jax.experimental import pallas as pl
from jax.experimental.pallas import tpu as pltpu
from jax.experimental.pallas import tpu_sc as plsc

_LANES = 16
_RB = 64


@functools.lru_cache(maxsize=None)
def _permute_cols(R, C, dtype_name):
    dtype = jnp.dtype(dtype_name)
    info = plsc.get_sparse_core_info()
    NC, NS = info.num_cores, info.num_subcores
    NW = NC * NS
    assert R % (NW * _RB) == 0
    assert C % _LANES == 0
    rows_per_w = R // NW
    n_blocks = rows_per_w // _RB
    n_cgrp = C // _LANES
    mesh = plsc.VectorSubcoreMesh(core_axis_name="c", subcore_axis_name="s")

    assert n_blocks % 2 == 0

    @functools.partial(
        pl.kernel,
        mesh=mesh,
        out_type=jax.ShapeDtypeStruct((R, C), dtype),
        compiler_params=pltpu.CompilerParams(needs_layout_passes=False),
        scratch_types=[
            pltpu.VMEM((C,), jnp.int32),
            pltpu.VMEM((_RB, C), dtype),
            pltpu.VMEM((_RB, C), dtype),
            pltpu.VMEM((_RB, C), dtype),
            pltpu.VMEM((_RB, C), dtype),
            pltpu.SemaphoreType.DMA,
            pltpu.SemaphoreType.DMA,
            pltpu.SemaphoreType.DMA,
            pltpu.SemaphoreType.DMA,
        ],
    )
    def k(x_hbm, perm_hbm, out_hbm, perm_v,
          ibuf0, ibuf1, obuf0, obuf1, si0, si1, so0, so1):
        wid = lax.axis_index("s") * NC + lax.axis_index("c")
        base = wid * rows_per_w
        pltpu.sync_copy(perm_hbm, perm_v)
        pvecs = [perm_v[pl.ds(j * _LANES, _LANES)] for j in range(n_cgrp)]
        ibufs, sis = (ibuf0, ibuf1), (si0, si1)
        obufs, sos = (obuf0, obuf1), (so0, so1)

        def start_in(i, buf, sem):
            pltpu.make_async_copy(
                x_hbm.at[pl.ds(base + i * _RB, _RB)], buf, sem).start()

        def compute(buf_in, buf_out):
            def row(r, c2):
                rvec = jnp.broadcast_to(r, (_LANES,)).astype(jnp.int32)
                for j in range(n_cgrp):
                    g = plsc.load_gather(buf_in, [rvec, pvecs[j]])
                    buf_out[r, pl.ds(j * _LANES, _LANES)] = g
                return c2

            lax.fori_loop(0, _RB, row, 0)

        start_in(0, ibufs[0], sis[0])

        def pair(jp, carry):
            for b in range(2):
                i = 2 * jp + b
                @pl.when(i + 1 < n_blocks)
                def _():
                    start_in(i + 1, ibufs[1 - b], sis[1 - b])

                pltpu.make_async_copy(
                    x_hbm.at[pl.ds(base, _RB)], ibufs[b], sis[b]).wait()
                @pl.when(i >= 2)
                def _():
                    pltpu.make_async_copy(
                        obufs[b], out_hbm.at[pl.ds(base, _RB)], sos[b]).wait()

                compute(ibufs[b], obufs[b])
                pltpu.make_async_copy(
                    obufs[b], out_hbm.at[pl.ds(base + i * _RB, _RB)],
                    sos[b]).start()
            return carry

        lax.fori_loop(0, n_blocks // 2, pair, 0)
        for b in range(2):
            pltpu.make_async_copy(
                obufs[b], out_hbm.at[pl.ds(base, _RB)], sos[b]).wait()

    return k


def kernel(x, permutation):
    b, c, h, w = x.shape
    xt = jnp.transpose(x, (0, 2, 3, 1)).reshape(b * h * w, c)
    zt = _permute_cols(b * h * w, c, x.dtype.name)(xt, permutation)
    z = jnp.transpose(zt.reshape(b, h, w, c), (0, 3, 1, 2))
    ldj = jnp.zeros((b,), x.dtype)
    return (z, ldj)
